# baseline (speedup 1.0000x reference)
import functools

import jax
import jax.numpy as jnp
from jax import lax
from jax.experimental import pallas as pl
from jax.experimental.pallas import tpu as pltpu
from jax.experimental.pallas import tpu_sc as plsc

NUM_TX = 4
NUM_STREAMS = 2
NUM_OFDM = 14
FFT = 4096
NUM_DATA = 12
BATCH = 64
LANE = 128
FB = FFT // LANE
TASKS_PER_BATCH_FACTOR = NUM_TX

_NPIECE = 4
QROWS = NUM_OFDM * FB // _NPIECE
_NBUF = _NPIECE

_SEGMENTS = (
    (0, 64, "d", 0),
    (64, 96, "p", 0),
    (96, 352, "d", 64),
    (352, 384, "p", 1),
    (384, 448, "d", 320),
)


def _piece_plan(q):
    lo, hi = q * QROWS, (q + 1) * QROWS
    plan = []
    for s_lo, s_hi, kind, src in _SEGMENTS:
        a, b = max(lo, s_lo), min(hi, s_hi)
        if a >= b:
            continue
        if kind == "p":
            assert a == s_lo and b == s_hi, "pilot plane must not straddle pieces"
            plan.append(("p", src, a - lo, b - a))
        else:
            plan.append(("d", src + (a - s_lo), a - lo, b - a))
    return tuple(plan)


_PIECE_PLANS = tuple(_piece_plan(q) for q in range(_NPIECE))


def _body(x_hbm, p_hbm, out_hbm, *scr):
    shared, vm0, vm1 = scr[0], scr[1], scr[2]
    gsems = scr[3 : 3 + _NBUF]
    ssems = scr[3 + _NBUF :]
    sid = lax.axis_index("s")
    bufs = (shared.at[sid, 0], vm0, shared.at[sid, 1], vm1)

    tasks_per_batch = TASKS_PER_BATCH_FACTOR * _NPIECE
    num_tasks = BATCH * tasks_per_batch
    info = plsc.get_sparse_core_info()
    nw = info.num_cores * info.num_subcores
    per_w = num_tasks // nw
    wid = lax.axis_index("s") * info.num_cores + lax.axis_index("c")
    base = wid * per_w

    def task_coords(i):
        g = base + i
        b = lax.div(g, tasks_per_batch)
        r = lax.rem(g, tasks_per_batch)
        t = lax.div(r, _NPIECE)
        return b, t

    def fire_gathers(i):
        b, t = task_coords(i)
        buf = bufs[i % _NBUF]
        sem = gsems[i % _NBUF]
        cs = []
        for kind, src0, dst0, n in _PIECE_PLANS[i % _NPIECE]:
            if kind == "d":
                src = x_hbm.at[b, t, pl.ds(src0, n), :, :]
            else:
                src = p_hbm.at[t, src0]
            cs.append(pltpu.async_copy(src, buf.at[pl.ds(dst0, n)], sem))
        return cs

    def fire_scatter(i):
        b, t = task_coords(i)
        return pltpu.async_copy(
            bufs[i % _NBUF],
            out_hbm.at[b, t, pl.ds((i % _NPIECE) * QROWS, QROWS), :, :],
            ssems[i % _NBUF],
        )

    gather_handles = [None] * per_w
    scatter_handles = [None] * per_w
    for k in range(_NBUF - 1):
        gather_handles[k] = fire_gathers(k)
    for i in range(per_w):
        j = i + _NBUF - 1
        if j < per_w:
            if i >= 1:
                scatter_handles[i - 1].wait()
            gather_handles[j] = fire_gathers(j)
        for c in gather_handles[i]:
            c.wait()
        scatter_handles[i] = fire_scatter(i)
    for i in range(per_w - _NBUF, per_w):
        scatter_handles[i].wait()


def kernel(x, pilots):
    xp = x.reshape(BATCH, NUM_TX, NUM_STREAMS, NUM_DATA * FB, LANE).transpose(
        0, 1, 3, 2, 4
    )
    pp = pilots.reshape(NUM_TX, NUM_STREAMS, 2, FB, LANE).transpose(
        0, 2, 3, 1, 4
    )

    mesh = plsc.VectorSubcoreMesh(core_axis_name="c", subcore_axis_name="s")
    run = functools.partial(
        pl.kernel,
        mesh=mesh,
        out_type=jax.ShapeDtypeStruct(
            (BATCH, NUM_TX, NUM_OFDM * FB, NUM_STREAMS, LANE), jnp.float32
        ),
        scratch_types=(
            [
                pltpu.VMEM_SHARED(
                    (16, 2, QROWS, NUM_STREAMS, LANE), jnp.float32
                ),
                pltpu.VMEM((QROWS, NUM_STREAMS, LANE), jnp.float32),
                pltpu.VMEM((QROWS, NUM_STREAMS, LANE), jnp.float32),
            ]
            + [pltpu.SemaphoreType.DMA] * (2 * _NBUF)
        ),
    )(_body)
    out = run(xp, pp)
    return (
        out.reshape(BATCH, NUM_TX, NUM_OFDM, FB, NUM_STREAMS, LANE)
        .transpose(0, 1, 4, 2, 3, 5)
        .reshape(BATCH, NUM_TX, NUM_STREAMS, NUM_OFDM, FFT)
    )


# speedup vs baseline: 1.0682x; 1.0682x over previous
import functools

import jax
import jax.numpy as jnp
from jax import lax
from jax.experimental import pallas as pl
from jax.experimental.pallas import tpu as pltpu
from jax.experimental.pallas import tpu_sc as plsc

NUM_TX = 4
NUM_STREAMS = 2
NUM_OFDM = 14
FFT = 4096
NUM_DATA = 12
BATCH = 64
LANE = 128
FB = FFT // LANE
TASKS_PER_BATCH_FACTOR = NUM_TX

_NPIECE = 2
QROWS = NUM_OFDM * FB // _NPIECE
_NBUF = _NPIECE

_SEGMENTS = (
    (0, 64, "d", 0),
    (64, 96, "p", 0),
    (96, 352, "d", 64),
    (352, 384, "p", 1),
    (384, 448, "d", 320),
)


def _piece_plan(q):
    lo, hi = q * QROWS, (q + 1) * QROWS
    plan = []
    for s_lo, s_hi, kind, src in _SEGMENTS:
        a, b = max(lo, s_lo), min(hi, s_hi)
        if a >= b:
            continue
        if kind == "p":
            assert a == s_lo and b == s_hi, "pilot plane must not straddle pieces"
            plan.append(("p", src, a - lo, b - a))
        else:
            plan.append(("d", src + (a - s_lo), a - lo, b - a))
    return tuple(plan)


_PIECE_PLANS = tuple(_piece_plan(q) for q in range(_NPIECE))


def _body(x_hbm, p_hbm, out_hbm, *scr):
    shared = scr[0]
    gsems = scr[1 : 1 + _NBUF]
    ssems = scr[1 + _NBUF :]
    sid = lax.axis_index("s")
    bufs = tuple(shared.at[sid, k] for k in range(_NBUF))

    tasks_per_batch = TASKS_PER_BATCH_FACTOR * _NPIECE
    num_tasks = BATCH * tasks_per_batch
    info = plsc.get_sparse_core_info()
    nw = info.num_cores * info.num_subcores
    per_w = num_tasks // nw
    wid = lax.axis_index("s") * info.num_cores + lax.axis_index("c")
    base = wid * per_w

    def task_coords(i):
        g = base + i
        b = lax.div(g, tasks_per_batch)
        r = lax.rem(g, tasks_per_batch)
        t = lax.div(r, _NPIECE)
        return b, t

    def fire_gathers(i):
        b, t = task_coords(i)
        buf = bufs[i % _NBUF]
        sem = gsems[i % _NBUF]
        cs = []
        for kind, src0, dst0, n in _PIECE_PLANS[i % _NPIECE]:
            if kind == "d":
                src = x_hbm.at[b, t, pl.ds(src0, n), :, :]
            else:
                src = p_hbm.at[t, src0]
            cs.append(pltpu.async_copy(src, buf.at[pl.ds(dst0, n)], sem))
        return cs

    def fire_scatter(i):
        b, t = task_coords(i)
        return pltpu.async_copy(
            bufs[i % _NBUF],
            out_hbm.at[b, t, pl.ds((i % _NPIECE) * QROWS, QROWS), :, :],
            ssems[i % _NBUF],
        )

    gather_handles = [None] * per_w
    scatter_handles = [None] * per_w
    for k in range(_NBUF - 1):
        gather_handles[k] = fire_gathers(k)
    for i in range(per_w):
        j = i + _NBUF - 1
        if j < per_w:
            if i >= 1:
                scatter_handles[i - 1].wait()
            gather_handles[j] = fire_gathers(j)
        for c in gather_handles[i]:
            c.wait()
        scatter_handles[i] = fire_scatter(i)
    for i in range(per_w - _NBUF, per_w):
        scatter_handles[i].wait()


def kernel(x, pilots):
    xp = x.reshape(BATCH, NUM_TX, NUM_STREAMS, NUM_DATA * FB, LANE).transpose(
        0, 1, 3, 2, 4
    )
    pp = pilots.reshape(NUM_TX, NUM_STREAMS, 2, FB, LANE).transpose(
        0, 2, 3, 1, 4
    )

    mesh = plsc.VectorSubcoreMesh(core_axis_name="c", subcore_axis_name="s")
    run = functools.partial(
        pl.kernel,
        mesh=mesh,
        out_type=jax.ShapeDtypeStruct(
            (BATCH, NUM_TX, NUM_OFDM * FB, NUM_STREAMS, LANE), jnp.float32
        ),
        scratch_types=(
            [
                pltpu.VMEM_SHARED(
                    (16, _NBUF, QROWS, NUM_STREAMS, LANE), jnp.float32
                )
            ]
            + [pltpu.SemaphoreType.DMA] * (2 * _NBUF)
        ),
    )(_body)
    out = run(xp, pp)
    return (
        out.reshape(BATCH, NUM_TX, NUM_OFDM, FB, NUM_STREAMS, LANE)
        .transpose(0, 1, 4, 2, 3, 5)
        .reshape(BATCH, NUM_TX, NUM_STREAMS, NUM_OFDM, FFT)
    )
